# Initial kernel scaffold; baseline (speedup 1.0000x reference)
import functools

import jax
import jax.numpy as jnp
from jax.experimental import pallas as pl
from jax.experimental.pallas import tpu as pltpu

_NEG_INF = float("-inf")


def _sim_kernel(src_ref, tar_ref, out_ref):
    a = src_ref[...]
    b = tar_ref[...]
    an = a / jnp.maximum(jnp.sqrt(jnp.sum(a * a, axis=1, keepdims=True)), 1e-12)
    bn = b / jnp.maximum(jnp.sqrt(jnp.sum(b * b, axis=1, keepdims=True)), 1e-12)
    prod = jax.lax.dot_general(an, bn, (((1,), (1,)), ((), ())),
                               preferred_element_type=jnp.float32)
    out_ref[...] = 0.5 * (prod + 1.0)


def _stats_kernel(sim_ref, labr_ref, labc_ref, scores_ref, asg_ref, *,
                  top_n_sim, ranking_k, n_src, cb, cpad):
    S = sim_ref[...]
    lab_row = labr_ref[...]
    lab_col = labc_ref[...]
    work = S
    for _ in range(top_n_sim):
        m = jnp.max(work, axis=0, keepdims=True)
        work = jnp.where(work == m, -1.0, work)
    topmask = (work == -1.0).astype(jnp.float32)
    ci_row = jax.lax.broadcasted_iota(jnp.int32, (cpad, n_src), 0)
    onehot_t = (ci_row == lab_row).astype(jnp.float32)
    counts = jax.lax.dot_general(onehot_t, topmask, (((1,), (0,)), ((), ())),
                                 preferred_element_type=jnp.float32)
    cmax = jnp.max(counts, axis=0, keepdims=True)
    ci = jax.lax.broadcasted_iota(jnp.int32, (cpad, cb), 0)
    asg = jnp.min(jnp.where(counts == cmax, ci, 10**6), axis=0, keepdims=True)
    same = lab_col == asg
    n_same = jnp.sum(same.astype(jnp.float32), axis=0, keepdims=True)
    w1 = jnp.where(same, S, -1.0)
    w2 = jnp.where(same, -1.0, S)
    s_same = jnp.zeros((1, cb), jnp.float32)
    s_diff = jnp.zeros((1, cb), jnp.float32)
    for _ in range(ranking_k):
        m1 = jnp.max(w1, axis=0, keepdims=True)
        s_same = s_same + m1
        w1 = jnp.where(w1 == m1, -1.0, w1)
        m2 = jnp.max(w2, axis=0, keepdims=True)
        s_diff = s_diff + m2
        w2 = jnp.where(w2 == m2, -1.0, w2)
    score = s_same / s_diff
    score = jnp.where(n_same >= ranking_k, score, _NEG_INF)
    score = jnp.where((n_src - n_same) >= ranking_k, score, -0.0)
    scores_ref[...] = score
    asg_ref[...] = asg.astype(jnp.int32)


def _rank_kernel(s_ref, st_ref, sel_ref, *, n_tgt, tb, top_ranked_n):
    s = s_ref[...]
    st = st_ref[...]
    j = jax.lax.broadcasted_iota(jnp.int32, (tb, n_tgt), 1)
    t = (jax.lax.broadcasted_iota(jnp.int32, (tb, n_tgt), 0)
         + pl.program_id(0) * tb)
    gt = (s > st).astype(jnp.float32)
    eqless = ((s == st) & (j < t)).astype(jnp.float32)
    rank = jnp.sum(gt + eqless, axis=1, keepdims=True)
    sel_ref[...] = (rank < top_ranked_n).astype(jnp.float32)


def _loss_kernel(sim_ref, lab_ref, asg_ref, sel_ref, out_ref, acc):
    i = pl.program_id(0)
    S = sim_ref[...]
    lab = lab_ref[...]
    asg = asg_ref[...]
    sel = sel_ref[...] > 0.0
    same = (lab == asg) & sel
    m = jnp.max(jnp.where(sel, S, _NEG_INF), axis=1, keepdims=True)
    e = jnp.where(sel, jnp.exp(S - m), 0.0)
    den = jnp.sum(e, axis=1, keepdims=True)
    num = jnp.sum(jnp.where(same, e, 0.0), axis=1, keepdims=True)
    nsame = jnp.sum(same.astype(jnp.float32), axis=1, keepdims=True)
    nsel = jnp.sum(sel.astype(jnp.float32))
    valid = (nsame > 0.0) & (nsame < nsel)
    contrib = jnp.where(valid, jnp.log(num / den), 0.0)
    psum = jnp.sum(contrib)
    pval = jnp.sum(valid.astype(jnp.float32))

    @pl.when(i == 0)
    def _():
        acc[0] = 0.0
        acc[1] = 0.0

    acc[0] += psum
    acc[1] += pval

    @pl.when(i == pl.num_programs(0) - 1)
    def _():
        out_ref[0, 0] = -acc[0] / jnp.maximum(acc[1], 1.0)


def kernel(output, src_labels):
    n = output.shape[0] // 2
    d = output.shape[1]
    n_src = n
    n_tgt = n
    top_n_sim = 10
    ranking_k = 5
    top_ranked_n = 2048
    cpad = 128

    out_src = output[:n]
    out_tar = output[n:]
    lab_row = src_labels.reshape(1, n_src)
    lab_col = src_labels.reshape(n_src, 1)

    bm = bn = 256
    sim = pl.pallas_call(
        _sim_kernel,
        grid=(n_src // bm, n_tgt // bn),
        in_specs=[
            pl.BlockSpec((bm, d), lambda i, j: (i, 0)),
            pl.BlockSpec((bn, d), lambda i, j: (j, 0)),
        ],
        out_specs=pl.BlockSpec((bm, bn), lambda i, j: (i, j)),
        out_shape=jax.ShapeDtypeStruct((n_src, n_tgt), jnp.float32),
    )(out_src, out_tar)

    cb = 128
    stats = functools.partial(_stats_kernel, top_n_sim=top_n_sim,
                              ranking_k=ranking_k, n_src=n_src, cb=cb,
                              cpad=cpad)
    scores, assigned = pl.pallas_call(
        stats,
        grid=(n_tgt // cb,),
        in_specs=[
            pl.BlockSpec((n_src, cb), lambda j: (0, j)),
            pl.BlockSpec((1, n_src), lambda j: (0, 0)),
            pl.BlockSpec((n_src, 1), lambda j: (0, 0)),
        ],
        out_specs=[
            pl.BlockSpec((1, cb), lambda j: (0, j)),
            pl.BlockSpec((1, cb), lambda j: (0, j)),
        ],
        out_shape=[
            jax.ShapeDtypeStruct((1, n_tgt), jnp.float32),
            jax.ShapeDtypeStruct((1, n_tgt), jnp.int32),
        ],
    )(sim, lab_row, lab_col)

    tb = 256
    scores_t = scores.reshape(n_tgt, 1)
    rank = functools.partial(_rank_kernel, n_tgt=n_tgt, tb=tb,
                             top_ranked_n=top_ranked_n)
    sel = pl.pallas_call(
        rank,
        grid=(n_tgt // tb,),
        in_specs=[
            pl.BlockSpec((1, n_tgt), lambda j: (0, 0)),
            pl.BlockSpec((tb, 1), lambda j: (j, 0)),
        ],
        out_specs=pl.BlockSpec((tb, 1), lambda j: (j, 0)),
        out_shape=jax.ShapeDtypeStruct((n_tgt, 1), jnp.float32),
    )(scores, scores_t)
    sel_row = sel.reshape(1, n_tgt)

    rb = 256
    loss = pl.pallas_call(
        _loss_kernel,
        grid=(n_src // rb,),
        in_specs=[
            pl.BlockSpec((rb, n_tgt), lambda i: (i, 0)),
            pl.BlockSpec((rb, 1), lambda i: (i, 0)),
            pl.BlockSpec((1, n_tgt), lambda i: (0, 0)),
            pl.BlockSpec((1, n_tgt), lambda i: (0, 0)),
        ],
        out_specs=pl.BlockSpec((1, 1), lambda i: (0, 0)),
        out_shape=jax.ShapeDtypeStruct((1, 1), jnp.float32),
        scratch_shapes=[pltpu.SMEM((2,), jnp.float32)],
    )(sim, lab_col, assigned, sel_row)

    return loss[0, 0]


# speedup vs baseline: 9.7926x; 9.7926x over previous
import functools

import jax
import jax.numpy as jnp
from jax.experimental import pallas as pl
from jax.experimental.pallas import tpu as pltpu

_NEG_INF = float("-inf")


def _sim_kernel(src_ref, tar_ref, out_ref):
    a = src_ref[...]
    b = tar_ref[...]
    an = a / jnp.maximum(jnp.sqrt(jnp.sum(a * a, axis=1, keepdims=True)), 1e-12)
    bn = b / jnp.maximum(jnp.sqrt(jnp.sum(b * b, axis=1, keepdims=True)), 1e-12)
    prod = jax.lax.dot_general(an, bn, (((1,), (1,)), ((), ())),
                               preferred_element_type=jnp.float32)
    out_ref[...] = 0.5 * (prod + 1.0)


def _stats_kernel(sim_ref, labr_ref, labc_ref, scores_ref, asg_ref, *,
                  top_n_sim, ranking_k, n_src, cb, cpad):
    S = sim_ref[...]
    lab_row = labr_ref[...]
    lab_col = labc_ref[...]
    work = S
    for _ in range(top_n_sim):
        m = jnp.max(work, axis=0, keepdims=True)
        work = jnp.where(work == m, -1.0, work)
    topmask = (work == -1.0).astype(jnp.float32)
    ci_row = jax.lax.broadcasted_iota(jnp.int32, (cpad, n_src), 0)
    onehot_t = (ci_row == lab_row).astype(jnp.float32)
    counts = jax.lax.dot_general(onehot_t, topmask, (((1,), (0,)), ((), ())),
                                 preferred_element_type=jnp.float32)
    cmax = jnp.max(counts, axis=0, keepdims=True)
    ci = jax.lax.broadcasted_iota(jnp.int32, (cpad, cb), 0)
    asg = jnp.min(jnp.where(counts == cmax, ci, 10**6), axis=0, keepdims=True)
    same = lab_col == asg
    n_same = jnp.sum(same.astype(jnp.float32), axis=0, keepdims=True)
    w1 = jnp.where(same, S, -1.0)
    w2 = jnp.where(same, -1.0, S)
    s_same = jnp.zeros((1, cb), jnp.float32)
    s_diff = jnp.zeros((1, cb), jnp.float32)
    for _ in range(ranking_k):
        m1 = jnp.max(w1, axis=0, keepdims=True)
        s_same = s_same + m1
        w1 = jnp.where(w1 == m1, -1.0, w1)
        m2 = jnp.max(w2, axis=0, keepdims=True)
        s_diff = s_diff + m2
        w2 = jnp.where(w2 == m2, -1.0, w2)
    score = s_same / s_diff
    score = jnp.where(n_same >= ranking_k, score, _NEG_INF)
    score = jnp.where((n_src - n_same) >= ranking_k, score, -0.0)
    scores_ref[...] = score
    asg_ref[...] = asg.astype(jnp.int32)


def _rank_kernel(s_ref, st_ref, sel_ref, *, n_tgt, tb, top_ranked_n):
    s = s_ref[...]
    st = st_ref[...]
    j = jax.lax.broadcasted_iota(jnp.int32, (tb, n_tgt), 1)
    t = (jax.lax.broadcasted_iota(jnp.int32, (tb, n_tgt), 0)
         + pl.program_id(0) * tb)
    gt = (s > st).astype(jnp.float32)
    eqless = ((s == st) & (j < t)).astype(jnp.float32)
    rank = jnp.sum(gt + eqless, axis=1, keepdims=True)
    sel_ref[...] = (rank < top_ranked_n).astype(jnp.float32)


def _loss_kernel(sim_ref, lab_ref, asg_ref, sel_ref, out_ref, acc):
    i = pl.program_id(0)
    S = sim_ref[...]
    lab = lab_ref[...]
    asg = asg_ref[...]
    sel = sel_ref[...] > 0.0
    same = (lab == asg) & sel
    m = jnp.max(jnp.where(sel, S, _NEG_INF), axis=1, keepdims=True)
    e = jnp.where(sel, jnp.exp(S - m), 0.0)
    den = jnp.sum(e, axis=1, keepdims=True)
    num = jnp.sum(jnp.where(same, e, 0.0), axis=1, keepdims=True)
    nsame = jnp.sum(same.astype(jnp.float32), axis=1, keepdims=True)
    nsel = jnp.sum(sel.astype(jnp.float32))
    valid = (nsame > 0.0) & (nsame < nsel)
    contrib = jnp.where(valid, jnp.log(num / den), 0.0)
    psum = jnp.sum(contrib)
    pval = jnp.sum(valid.astype(jnp.float32))

    @pl.when(i == 0)
    def _():
        acc[0] = 0.0
        acc[1] = 0.0

    acc[0] += psum
    acc[1] += pval

    @pl.when(i == pl.num_programs(0) - 1)
    def _():
        val = -acc[0] / jnp.maximum(acc[1], 1.0)
        out_ref[...] = jnp.full((1, 1), val, jnp.float32)


def kernel(output, src_labels):
    n = output.shape[0] // 2
    d = output.shape[1]
    n_src = n
    n_tgt = n
    top_n_sim = 10
    ranking_k = 5
    top_ranked_n = 2048
    cpad = 128

    out_src = output[:n]
    out_tar = output[n:]
    lab_row = src_labels.reshape(1, n_src)
    lab_col = src_labels.reshape(n_src, 1)

    bm = bn = 256
    sim = pl.pallas_call(
        _sim_kernel,
        grid=(n_src // bm, n_tgt // bn),
        in_specs=[
            pl.BlockSpec((bm, d), lambda i, j: (i, 0)),
            pl.BlockSpec((bn, d), lambda i, j: (j, 0)),
        ],
        out_specs=pl.BlockSpec((bm, bn), lambda i, j: (i, j)),
        out_shape=jax.ShapeDtypeStruct((n_src, n_tgt), jnp.float32),
    )(out_src, out_tar)

    cb = 128
    stats = functools.partial(_stats_kernel, top_n_sim=top_n_sim,
                              ranking_k=ranking_k, n_src=n_src, cb=cb,
                              cpad=cpad)
    scores, assigned = pl.pallas_call(
        stats,
        grid=(n_tgt // cb,),
        in_specs=[
            pl.BlockSpec((n_src, cb), lambda j: (0, j)),
            pl.BlockSpec((1, n_src), lambda j: (0, 0)),
            pl.BlockSpec((n_src, 1), lambda j: (0, 0)),
        ],
        out_specs=[
            pl.BlockSpec((1, cb), lambda j: (0, j)),
            pl.BlockSpec((1, cb), lambda j: (0, j)),
        ],
        out_shape=[
            jax.ShapeDtypeStruct((1, n_tgt), jnp.float32),
            jax.ShapeDtypeStruct((1, n_tgt), jnp.int32),
        ],
    )(sim, lab_row, lab_col)

    tb = 256
    scores_t = scores.reshape(n_tgt, 1)
    rank = functools.partial(_rank_kernel, n_tgt=n_tgt, tb=tb,
                             top_ranked_n=top_ranked_n)
    sel = pl.pallas_call(
        rank,
        grid=(n_tgt // tb,),
        in_specs=[
            pl.BlockSpec((1, n_tgt), lambda j: (0, 0)),
            pl.BlockSpec((tb, 1), lambda j: (j, 0)),
        ],
        out_specs=pl.BlockSpec((tb, 1), lambda j: (j, 0)),
        out_shape=jax.ShapeDtypeStruct((n_tgt, 1), jnp.float32),
    )(scores, scores_t)
    sel_row = sel.reshape(1, n_tgt)

    rb = 256
    loss = pl.pallas_call(
        _loss_kernel,
        grid=(n_src // rb,),
        in_specs=[
            pl.BlockSpec((rb, n_tgt), lambda i: (i, 0)),
            pl.BlockSpec((rb, 1), lambda i: (i, 0)),
            pl.BlockSpec((1, n_tgt), lambda i: (0, 0)),
            pl.BlockSpec((1, n_tgt), lambda i: (0, 0)),
        ],
        out_specs=pl.BlockSpec((1, 1), lambda i: (0, 0)),
        out_shape=jax.ShapeDtypeStruct((1, 1), jnp.float32),
        scratch_shapes=[pltpu.SMEM((2,), jnp.float32)],
    )(sim, lab_col, assigned, sel_row)

    return loss[0, 0]
